# baseline (speedup 1.0000x reference)
import functools

import jax
import jax.numpy as jnp
from jax import lax
from jax.experimental import pallas as pl
from jax.experimental.pallas import tpu as pltpu
from jax.experimental.pallas import tpu_sc as plsc

N = 10000
E = 320000
IN_FEAT = 128
OUT_FEAT = 16
ORDER = 3
K = OUT_FEAT * (ORDER - 1) + 1
KP = 64

NT = 16
NPAD = 10240
NODES_PT = NPAD // NT
EPT = E // NT
L = 16


def _prep_body(x_ref, row_ref, col_ref, z0_ref, pk_ref):
    z0_ref[...] = jnp.sum(x_ref[...], axis=1)
    pk_ref[...] = col_ref[...] | (row_ref[...] << 16)


def _prep(x, rows, cols):
    return pl.pallas_call(
        _prep_body,
        out_shape=(jax.ShapeDtypeStruct((N,), jnp.float32),
                   jax.ShapeDtypeStruct((E,), jnp.int32)),
    )(x, rows, cols)


def _sc_body(z0, packed, vals, o1a, o2a, c1a, c2a, out,
             pk_v, val_v, z_v, acc_a, acc_b, red_v, zsl_v, y_v,
             o1_v, o2_v, c1_v, c2_v, dma_sem, part_sh, z_sh):
    tid = lax.axis_index("s")
    eoff = tid * EPT
    noff = tid * NODES_PT

    pltpu.sync_copy(packed.at[pl.ds(eoff, EPT)], pk_v)
    pltpu.sync_copy(vals.at[pl.ds(eoff, EPT)], val_v)
    pltpu.sync_copy(o1a, o1_v)
    pltpu.sync_copy(o2a, o2_v)
    pltpu.sync_copy(c1a, c1_v)
    pltpu.sync_copy(c2a, c2_v)
    pltpu.sync_copy(z0, z_v.at[pl.ds(0, N)])

    zero = jnp.zeros((L,), jnp.float32)
    for i in range(N, NPAD, L):
        z_v[pl.ds(i, L)] = zero

    def _zero_y(i):
        y_v[pl.ds(i, L)] = zero

    plsc.parallel_loop(0, OUT_FEAT * NODES_PT, step=L, unroll=8)(_zero_y)

    accs = [acc_a, acc_b]

    def _zero_into(acc_v):
        def _zero_acc(i):
            acc_v[pl.ds(i, L)] = zero
        plsc.parallel_loop(0, NPAD, step=L, unroll=8)(_zero_acc)

    _zero_into(acc_a)

    def _edges_into(acc_v):
        def _edges(i):
            pk = pk_v[pl.ds(i, L)]
            ci = pk & 0xFFFF
            ri = lax.shift_right_logical(pk, 16)
            zg = plsc.load_gather(z_v, [ci])
            vv = val_v[pl.ds(i, L)]
            plsc.addupdate_scatter(acc_v, [ri], zg * vv)
        plsc.parallel_loop(0, EPT, step=L, unroll=16)(_edges)

    def _one_k(k, cur, nxt, zero_next):
        _edges_into(cur)
        cp = pltpu.async_copy(cur, part_sh.at[tid], dma_sem)
        if zero_next:
            _zero_into(nxt)
        cp.wait()
        plsc.subcore_barrier()
        HALF = 384
        REST = NODES_PT - HALF
        cpa = pltpu.async_copy(part_sh.at[:, pl.ds(noff, HALF)],
                               red_v.at[:, pl.ds(0, HALF)], dma_sem)
        cpb = pltpu.async_copy(part_sh.at[:, pl.ds(noff + HALF, REST)],
                               red_v.at[:, pl.ds(HALF, REST)], dma_sem)
        o1 = o1_v[pl.ds(k, L)][0]
        o2 = o2_v[pl.ds(k, L)][0]
        c1 = c1_v[pl.ds(k, L)][0]
        c2 = c2_v[pl.ds(k, L)][0]

        def _reduce(c):
            parts = [red_v[t, pl.ds(c, L)] for t in range(NT)]
            while len(parts) > 1:
                parts = [parts[a] + parts[a + 1]
                         for a in range(0, len(parts), 2)]
            zsl_v[pl.ds(c, L)] = parts[0]

        cpa.wait()
        plsc.parallel_loop(0, HALF, step=L, unroll=2)(_reduce)
        cpb.wait()
        plsc.parallel_loop(HALF, NODES_PT, step=L, unroll=2)(_reduce)
        pltpu.sync_copy(zsl_v, z_sh.at[pl.ds(noff, NODES_PT)])

        def _upd(c):
            s = zsl_v[pl.ds(c, L)]
            y_v[pl.ds(o1 + c, L)] = y_v[pl.ds(o1 + c, L)] + c1 * s
            y_v[pl.ds(o2 + c, L)] = y_v[pl.ds(o2 + c, L)] + c2 * s

        plsc.parallel_loop(0, NODES_PT, step=L, unroll=4)(_upd)
        plsc.subcore_barrier()
        pltpu.sync_copy(z_sh, z_v)

    def _two_k(m, carry):
        _one_k(2 * m, acc_a, acc_b, True)
        _one_k(2 * m + 1, acc_b, acc_a, True)
        return carry

    lax.fori_loop(0, K // 2, _two_k, 0)
    _one_k(K - 1, acc_a, acc_b, False)
    for j in range(OUT_FEAT):
        pltpu.sync_copy(y_v.at[pl.ds(j * NODES_PT, NODES_PT)],
                        out.at[j, pl.ds(noff, NODES_PT)])


def _sc_filter(z0, packed, vals, o1a, o2a, c1a, c2a):
    mesh = plsc.VectorSubcoreMesh(
        core_axis_name="c", subcore_axis_name="s", num_cores=1)
    return pl.kernel(
        _sc_body,
        out_type=jax.ShapeDtypeStruct((OUT_FEAT, NPAD), jnp.float32),
        mesh=mesh,
        compiler_params=pltpu.CompilerParams(needs_layout_passes=False),
        scratch_types=[
            pltpu.VMEM((EPT,), jnp.int32),
            pltpu.VMEM((EPT,), jnp.float32),
            pltpu.VMEM((NPAD,), jnp.float32),
            pltpu.VMEM((NPAD,), jnp.float32),
            pltpu.VMEM((NPAD,), jnp.float32),
            pltpu.VMEM((NT, NODES_PT), jnp.float32),
            pltpu.VMEM((NODES_PT,), jnp.float32),
            pltpu.VMEM((OUT_FEAT * NODES_PT,), jnp.float32),
            pltpu.VMEM((KP,), jnp.int32),
            pltpu.VMEM((KP,), jnp.int32),
            pltpu.VMEM((KP,), jnp.float32),
            pltpu.VMEM((KP,), jnp.float32),
            pltpu.SemaphoreType.DMA,
            pltpu.VMEM_SHARED((NT, NPAD), jnp.float32),
            pltpu.VMEM_SHARED((NPAD,), jnp.float32),
        ],
    )(z0, packed, vals, o1a, o2a, c1a, c2a)


@jax.jit
def kernel(x, edge_row, edge_col, edge_val, filterCoeff, ind):
    del ind
    vals = edge_val.astype(jnp.float32)

    karr = jnp.arange(1, K + 1, dtype=jnp.int32)
    j_hi = (karr - 1) // 2
    j_lo = j_hi - 1
    jh = jnp.clip(j_hi, 0, OUT_FEAT - 1)
    jl = jnp.clip(j_lo, 0, OUT_FEAT - 1)
    i_hi = karr - 1 - 2 * j_hi
    c_hi = jnp.where(j_hi <= OUT_FEAT - 1, filterCoeff[jh, i_hi], 0.0)
    c_lo = jnp.where((j_lo >= 0) & (i_hi == 0), filterCoeff[jl, ORDER - 1],
                     0.0)
    pad = KP - K
    o1a = jnp.pad(jh * NODES_PT, (0, pad)).astype(jnp.int32)
    o2a = jnp.pad(jl * NODES_PT, (0, pad)).astype(jnp.int32)
    c1a = jnp.pad(c_hi, (0, pad)).astype(jnp.float32)
    c2a = jnp.pad(c_lo, (0, pad)).astype(jnp.float32)

    z0, packed = _prep(x, edge_row.astype(jnp.int32),
                       edge_col.astype(jnp.int32))
    y = _sc_filter(z0, packed, vals, o1a, o2a, c1a, c2a)
    return y.T[:N]


# speedup vs baseline: 1.0059x; 1.0059x over previous
import jax
import jax.numpy as jnp
from jax import lax
from jax.experimental import pallas as pl
from jax.experimental.pallas import tpu as pltpu
from jax.experimental.pallas import tpu_sc as plsc

N = 10000
E = 320000
IN_FEAT = 128
OUT_FEAT = 16
ORDER = 3
K = OUT_FEAT * (ORDER - 1) + 1
KP = 64

NT = 16
NPAD = 10240
NODES_PT = NPAD // NT
EPT = E // NT
L = 16


def _prep_body(x_ref, row_ref, col_ref, z0_ref, pk_ref):
    z0_ref[...] = jnp.sum(x_ref[...], axis=1)
    pk_ref[...] = col_ref[...] | (row_ref[...] << 16)


def _prep(x, rows, cols):
    return pl.pallas_call(
        _prep_body,
        out_shape=(jax.ShapeDtypeStruct((N,), jnp.float32),
                   jax.ShapeDtypeStruct((E,), jnp.int32)),
    )(x, rows, cols)


def _sc_body(z0, packed, vals, o1a, o2a, c1a, c2a, out,
             pk_v, val_v, z_v, acc_a, acc_b, red_v, zsl_v, y_v,
             o1_v, o2_v, c1_v, c2_v, dma_sem, part_sh, z_sh):
    tid = lax.axis_index("s")
    eoff = tid * EPT
    noff = tid * NODES_PT

    pltpu.sync_copy(packed.at[pl.ds(eoff, EPT)], pk_v)
    pltpu.sync_copy(vals.at[pl.ds(eoff, EPT)], val_v)
    pltpu.sync_copy(o1a, o1_v)
    pltpu.sync_copy(o2a, o2_v)
    pltpu.sync_copy(c1a, c1_v)
    pltpu.sync_copy(c2a, c2_v)
    pltpu.sync_copy(z0, z_v.at[pl.ds(0, N)])

    zero = jnp.zeros((L,), jnp.float32)
    for i in range(N, NPAD, L):
        z_v[pl.ds(i, L)] = zero

    def _zero_y(i):
        y_v[pl.ds(i, L)] = zero

    plsc.parallel_loop(0, OUT_FEAT * NODES_PT, step=L, unroll=8)(_zero_y)

    accs = [acc_a, acc_b]

    def _zero_into(acc_v):
        def _zero_acc(i):
            acc_v[pl.ds(i, L)] = zero
        plsc.parallel_loop(0, NPAD, step=L, unroll=8)(_zero_acc)

    _zero_into(acc_a)

    def _edges_into(acc_v):
        def _edges(i):
            pk = pk_v[pl.ds(i, L)]
            ci = pk & 0xFFFF
            ri = lax.shift_right_logical(pk, 16)
            zg = plsc.load_gather(z_v, [ci])
            vv = val_v[pl.ds(i, L)]
            plsc.addupdate_scatter(acc_v, [ri], zg * vv)
        plsc.parallel_loop(0, EPT, step=L, unroll=25)(_edges)

    def _one_k(k, cur, nxt, zero_next):
        _edges_into(cur)
        cp = pltpu.async_copy(cur, part_sh.at[tid], dma_sem)
        if zero_next:
            _zero_into(nxt)
        cp.wait()
        plsc.subcore_barrier()
        HALF = 384
        REST = NODES_PT - HALF
        cpa = pltpu.async_copy(part_sh.at[:, pl.ds(noff, HALF)],
                               red_v.at[:, pl.ds(0, HALF)], dma_sem)
        cpb = pltpu.async_copy(part_sh.at[:, pl.ds(noff + HALF, REST)],
                               red_v.at[:, pl.ds(HALF, REST)], dma_sem)
        o1 = o1_v[pl.ds(k, L)][0]
        o2 = o2_v[pl.ds(k, L)][0]
        c1 = c1_v[pl.ds(k, L)][0]
        c2 = c2_v[pl.ds(k, L)][0]

        def _reduce(c):
            parts = [red_v[t, pl.ds(c, L)] for t in range(NT)]
            while len(parts) > 1:
                parts = [parts[a] + parts[a + 1]
                         for a in range(0, len(parts), 2)]
            zsl_v[pl.ds(c, L)] = parts[0]

        cpa.wait()
        plsc.parallel_loop(0, HALF, step=L, unroll=2)(_reduce)
        cpb.wait()
        plsc.parallel_loop(HALF, NODES_PT, step=L, unroll=2)(_reduce)
        pltpu.sync_copy(zsl_v, z_sh.at[pl.ds(noff, NODES_PT)])

        def _upd(c):
            s = zsl_v[pl.ds(c, L)]
            y_v[pl.ds(o1 + c, L)] = y_v[pl.ds(o1 + c, L)] + c1 * s
            y_v[pl.ds(o2 + c, L)] = y_v[pl.ds(o2 + c, L)] + c2 * s

        plsc.parallel_loop(0, NODES_PT, step=L, unroll=4)(_upd)
        plsc.subcore_barrier()
        pltpu.sync_copy(z_sh, z_v)

    def _two_k(m, carry):
        _one_k(2 * m, acc_a, acc_b, True)
        _one_k(2 * m + 1, acc_b, acc_a, True)
        return carry

    lax.fori_loop(0, K // 2, _two_k, 0)
    _one_k(K - 1, acc_a, acc_b, False)
    for j in range(OUT_FEAT):
        pltpu.sync_copy(y_v.at[pl.ds(j * NODES_PT, NODES_PT)],
                        out.at[j, pl.ds(noff, NODES_PT)])


def _sc_filter(z0, packed, vals, o1a, o2a, c1a, c2a):
    mesh = plsc.VectorSubcoreMesh(
        core_axis_name="c", subcore_axis_name="s", num_cores=1)
    return pl.kernel(
        _sc_body,
        out_type=jax.ShapeDtypeStruct((OUT_FEAT, NPAD), jnp.float32),
        mesh=mesh,
        compiler_params=pltpu.CompilerParams(needs_layout_passes=False),
        scratch_types=[
            pltpu.VMEM((EPT,), jnp.int32),
            pltpu.VMEM((EPT,), jnp.float32),
            pltpu.VMEM((NPAD,), jnp.float32),
            pltpu.VMEM((NPAD,), jnp.float32),
            pltpu.VMEM((NPAD,), jnp.float32),
            pltpu.VMEM((NT, NODES_PT), jnp.float32),
            pltpu.VMEM((NODES_PT,), jnp.float32),
            pltpu.VMEM((OUT_FEAT * NODES_PT,), jnp.float32),
            pltpu.VMEM((KP,), jnp.int32),
            pltpu.VMEM((KP,), jnp.int32),
            pltpu.VMEM((KP,), jnp.float32),
            pltpu.VMEM((KP,), jnp.float32),
            pltpu.SemaphoreType.DMA,
            pltpu.VMEM_SHARED((NT, NPAD), jnp.float32),
            pltpu.VMEM_SHARED((NPAD,), jnp.float32),
        ],
    )(z0, packed, vals, o1a, o2a, c1a, c2a)


@jax.jit
def kernel(x, edge_row, edge_col, edge_val, filterCoeff, ind):
    del ind
    vals = edge_val.astype(jnp.float32)

    karr = jnp.arange(1, K + 1, dtype=jnp.int32)
    j_hi = (karr - 1) // 2
    j_lo = j_hi - 1
    jh = jnp.clip(j_hi, 0, OUT_FEAT - 1)
    jl = jnp.clip(j_lo, 0, OUT_FEAT - 1)
    i_hi = karr - 1 - 2 * j_hi
    c_hi = jnp.where(j_hi <= OUT_FEAT - 1, filterCoeff[jh, i_hi], 0.0)
    c_lo = jnp.where((j_lo >= 0) & (i_hi == 0), filterCoeff[jl, ORDER - 1],
                     0.0)
    pad = KP - K
    o1a = jnp.pad(jh * NODES_PT, (0, pad)).astype(jnp.int32)
    o2a = jnp.pad(jl * NODES_PT, (0, pad)).astype(jnp.int32)
    c1a = jnp.pad(c_hi, (0, pad)).astype(jnp.float32)
    c2a = jnp.pad(c_lo, (0, pad)).astype(jnp.float32)

    z0, packed = _prep(x, edge_row.astype(jnp.int32),
                       edge_col.astype(jnp.int32))
    y = _sc_filter(z0, packed, vals, o1a, o2a, c1a, c2a)
    return y.T[:N]
